# baseline (speedup 1.0000x reference)
import functools

import jax
import jax.numpy as jnp
from jax import lax
from jax.experimental import pallas as pl
from jax.experimental.pallas import tpu as pltpu
from jax.experimental.pallas import tpu_sc as plsc

CHUNK = 512
NBUF = 2


@functools.lru_cache(maxsize=None)
def _make_gather(num_chunks: int, dim: int):
    info = plsc.get_sparse_core_info()
    nc, ns = info.num_cores, info.num_subcores
    nw = nc * ns
    num_groups = num_chunks // NBUF
    assert num_chunks == num_groups * NBUF
    mesh = plsc.VectorSubcoreMesh(core_axis_name="c", subcore_axis_name="s")

    @functools.partial(
        pl.kernel,
        out_type=jax.ShapeDtypeStruct((nw, num_chunks, CHUNK, dim), jnp.float32),
        mesh=mesh,
        scratch_types=[
            pltpu.VMEM((num_chunks, CHUNK), jnp.int32),
            pltpu.VMEM((NBUF, CHUNK, dim), jnp.float32),
            pltpu.SemaphoreType.DMA((NBUF,)),
            pltpu.SemaphoreType.DMA((NBUF,)),
        ],
        compiler_params=pltpu.CompilerParams(use_tc_tiling_on_sc=False),
    )
    def gather_kernel(idx_hbm, table_hbm, out_hbm, idx_v, bufs, gsem, ssem):
        wid = lax.axis_index("c") * ns + lax.axis_index("s")
        pltpu.sync_copy(idx_hbm.at[wid], idx_v)

        def gather(j, b):
            return pltpu.make_async_copy(
                table_hbm.at[idx_v.at[j]], bufs.at[b], gsem.at[b])

        def scatter(j, b):
            return pltpu.make_async_copy(
                bufs.at[b], out_hbm.at[wid, j], ssem.at[b])

        for b in range(NBUF):
            gather(b, b).start()

        def body(g, carry):
            base = g * NBUF
            for b in range(NBUF):
                gather(base + b, b).wait()
                scatter(base + b, b).start()
            for b in range(NBUF):
                scatter(base + b, b).wait()
                gather(base + NBUF + b, b).start()
            return carry

        lax.fori_loop(0, num_groups - 1, body, 0)

        base = (num_groups - 1) * NBUF
        for b in range(NBUF):
            gather(base + b, b).wait()
            scatter(base + b, b).start()
        for b in range(NBUF):
            scatter(base + b, b).wait()

    return gather_kernel


def kernel(token_ids, weight):
    orig_shape = token_ids.shape
    dim = weight.shape[1]
    idx = token_ids.reshape(-1).astype(jnp.int32)
    b = idx.shape[0]
    info = plsc.get_sparse_core_info()
    nw = info.num_cores * info.num_subcores
    num_chunks = b // (nw * CHUNK)
    assert b == nw * num_chunks * CHUNK, (b, nw, CHUNK)
    idx3 = idx.reshape(nw, num_chunks, CHUNK)
    out = _make_gather(num_chunks, dim)(idx3, weight)
    return out.reshape(*orig_shape, dim)


# speedup vs baseline: 1.0024x; 1.0024x over previous
import functools

import jax
import jax.numpy as jnp
from jax import lax
from jax.experimental import pallas as pl
from jax.experimental.pallas import tpu as pltpu
from jax.experimental.pallas import tpu_sc as plsc

LANES = 16
CHUNK = 128
NBUF = 8


@functools.lru_cache(maxsize=None)
def _make_gather(num_chunks: int, dim: int):
    info = plsc.get_sparse_core_info()
    nc, ns = info.num_cores, info.num_subcores
    nw = nc * ns
    num_groups = num_chunks // NBUF
    assert num_chunks == num_groups * NBUF
    vecs = CHUNK // LANES
    mesh = plsc.VectorSubcoreMesh(core_axis_name="c", subcore_axis_name="s")

    @functools.partial(
        pl.kernel,
        out_type=jax.ShapeDtypeStruct((nw, num_chunks, CHUNK, dim), jnp.float32),
        mesh=mesh,
        scratch_types=[
            pltpu.VMEM((num_chunks, CHUNK), jnp.int32),
            pltpu.VMEM((NBUF, CHUNK, dim), jnp.float32),
            pltpu.SemaphoreType.DMA((NBUF,)),
            pltpu.SemaphoreType.DMA((NBUF,)),
        ],
        compiler_params=pltpu.CompilerParams(use_tc_tiling_on_sc=False),
    )
    def gather_kernel(idx_hbm, table_hbm, out_hbm, idx_v, bufs, gsem, ssem):
        wid = lax.axis_index("c") * ns + lax.axis_index("s")
        pltpu.sync_copy(idx_hbm.at[wid], idx_v)

        def row_gather(j, k, b):
            vec = idx_v[j, pl.ds(k * LANES, LANES)]
            return pltpu.make_async_copy(
                table_hbm.at[vec],
                bufs.at[b, pl.ds(k * LANES, LANES)],
                gsem.at[b])

        def start_gathers(j, b):
            def inner(k, carry):
                row_gather(j, k, b).start()
                return carry
            lax.fori_loop(0, vecs, inner, 0)

        def wait_gathers(j, b):
            def inner(k, carry):
                row_gather(j, k, b).wait()
                return carry
            lax.fori_loop(0, vecs, inner, 0)

        def scatter(j, b):
            return pltpu.make_async_copy(
                bufs.at[b], out_hbm.at[wid, j], ssem.at[b])

        for b in range(NBUF):
            start_gathers(b, b)

        def body(g, carry):
            base = g * NBUF
            for b in range(NBUF):
                wait_gathers(base + b, b)
                scatter(base + b, b).start()
            for b in range(NBUF):
                scatter(base + b, b).wait()
                start_gathers(base + NBUF + b, b)
            return carry

        lax.fori_loop(0, num_groups - 1, body, 0)

        base = (num_groups - 1) * NBUF
        for b in range(NBUF):
            wait_gathers(base + b, b)
            scatter(base + b, b).start()
        for b in range(NBUF):
            scatter(base + b, b).wait()

    return gather_kernel


def kernel(token_ids, weight):
    orig_shape = token_ids.shape
    dim = weight.shape[1]
    idx = token_ids.reshape(-1).astype(jnp.int32)
    b = idx.shape[0]
    info = plsc.get_sparse_core_info()
    nw = info.num_cores * info.num_subcores
    num_chunks = b // (nw * CHUNK)
    assert b == nw * num_chunks * CHUNK, (b, nw, CHUNK)
    idx3 = idx.reshape(nw, num_chunks, CHUNK)
    out = _make_gather(num_chunks, dim)(idx3, weight)
    return out.reshape(*orig_shape, dim)


# speedup vs baseline: 1.2148x; 1.2119x over previous
import functools

import jax
import jax.numpy as jnp
from jax import lax
from jax.experimental import pallas as pl
from jax.experimental.pallas import tpu as pltpu
from jax.experimental.pallas import tpu_sc as plsc

CHUNK = 128
NBUF = 4


@functools.lru_cache(maxsize=None)
def _make_gather(num_chunks: int, dim: int):
    info = plsc.get_sparse_core_info()
    nc, ns = info.num_cores, info.num_subcores
    nw = nc * ns
    num_groups = num_chunks // NBUF
    assert num_chunks == num_groups * NBUF
    mesh = plsc.VectorSubcoreMesh(core_axis_name="c", subcore_axis_name="s")

    @functools.partial(
        pl.kernel,
        out_type=jax.ShapeDtypeStruct((nw, num_chunks, CHUNK, dim), jnp.float32),
        mesh=mesh,
        scratch_types=[
            pltpu.VMEM((num_chunks, CHUNK), jnp.int32),
            pltpu.VMEM((NBUF, CHUNK, dim), jnp.float32),
            pltpu.SemaphoreType.DMA((NBUF,)),
            pltpu.SemaphoreType.DMA((NBUF,)),
        ],
    )
    def gather_kernel(idx_hbm, table_hbm, out_hbm, idx_v, bufs, gsem, ssem):
        wid = lax.axis_index("c") * ns + lax.axis_index("s")
        pltpu.sync_copy(idx_hbm.at[wid], idx_v)

        def start_gathers(j, b):
            def inner(k, carry):
                vec = idx_v[j, pl.ds(k * 16, 16)]
                for i in range(16):
                    pltpu.make_async_copy(
                        table_hbm.at[vec[i]],
                        bufs.at[b, k * 16 + i],
                        gsem.at[b]).start()
                return carry
            lax.fori_loop(0, CHUNK // 16, inner, 0)

        def wait_gathers(b):
            def inner(r, carry):
                pltpu.make_async_copy(
                    table_hbm.at[0], bufs.at[b, 0], gsem.at[b]).wait()
                return carry
            lax.fori_loop(0, CHUNK, inner, 0)

        def scatter(j, b):
            return pltpu.make_async_copy(
                bufs.at[b], out_hbm.at[wid, j], ssem.at[b])

        for b in range(NBUF):
            start_gathers(b, b)

        def body(g, carry):
            base = g * NBUF
            for b in range(NBUF):
                wait_gathers(b)
                scatter(base + b, b).start()
            for b in range(NBUF):
                scatter(base + b, b).wait()
                start_gathers(base + NBUF + b, b)
            return carry

        lax.fori_loop(0, num_groups - 1, body, 0)

        base = (num_groups - 1) * NBUF
        for b in range(NBUF):
            wait_gathers(b)
            scatter(base + b, b).start()
        for b in range(NBUF):
            scatter(base + b, b).wait()

    return gather_kernel


def kernel(token_ids, weight):
    orig_shape = token_ids.shape
    dim = weight.shape[1]
    idx = token_ids.reshape(-1).astype(jnp.int32)
    b = idx.shape[0]
    info = plsc.get_sparse_core_info()
    nw = info.num_cores * info.num_subcores
    num_chunks = b // (nw * CHUNK)
    assert b == nw * num_chunks * CHUNK, (b, nw, CHUNK)
    idx3 = idx.reshape(nw, num_chunks, CHUNK)
    out = _make_gather(num_chunks, dim)(idx3, weight)
    return out.reshape(*orig_shape, dim)


# speedup vs baseline: 1.4151x; 1.1649x over previous
import functools

import jax
import jax.numpy as jnp
from jax import lax
from jax.experimental import pallas as pl
from jax.experimental.pallas import tpu as pltpu
from jax.experimental.pallas import tpu_sc as plsc

TOK_CHUNK = 8
NBUF = 4


@functools.lru_cache(maxsize=None)
def _make_gather(num_tokens: int, seq: int, dim: int):
    info = plsc.get_sparse_core_info()
    nc, ns = info.num_cores, info.num_subcores
    nw = nc * ns
    tok_per_w = num_tokens // nw
    num_chunks = tok_per_w // TOK_CHUNK
    num_groups = num_chunks // NBUF
    assert num_chunks == num_groups * NBUF
    rows_per_chunk = TOK_CHUNK * seq
    rows_per_w = tok_per_w * seq
    assert rows_per_chunk % 16 == 0
    mesh = plsc.VectorSubcoreMesh(core_axis_name="c", subcore_axis_name="s")

    @functools.partial(
        pl.kernel,
        out_type=jax.ShapeDtypeStruct((num_tokens, seq, dim), jnp.float32),
        mesh=mesh,
        scratch_types=[
            pltpu.VMEM((rows_per_w // 128, 128), jnp.int32),
            pltpu.VMEM((NBUF, TOK_CHUNK, seq, dim), jnp.float32),
            pltpu.SemaphoreType.DMA((NBUF,)),
            pltpu.SemaphoreType.DMA((NBUF,)),
        ],
    )
    def gather_kernel(idx_hbm, table_hbm, out_hbm, idx_v, bufs, gsem, ssem):
        wid = lax.axis_index("c") * ns + lax.axis_index("s")
        pltpu.sync_copy(idx_hbm.at[wid], idx_v)

        def start_gathers(j, b):
            def inner(k, carry):
                flat = j * rows_per_chunk + k * 16
                vec = idx_v[flat // 128, pl.ds(lax.rem(flat, 128), 16)]
                for i in range(16):
                    r = flat + i
                    pltpu.make_async_copy(
                        table_hbm.at[vec[i]],
                        bufs.at[b, lax.rem(r, rows_per_chunk) // seq,
                                lax.rem(r, seq)],
                        gsem.at[b]).start()
                return carry
            lax.fori_loop(0, rows_per_chunk // 16, inner, 0)

        def wait_gathers(b):
            def inner(k, carry):
                pltpu.make_async_copy(
                    table_hbm.at[0], bufs.at[b, 0, 0], gsem.at[b]).wait()
                return carry
            lax.fori_loop(0, rows_per_chunk, inner, 0)

        def scatter(j, b):
            return pltpu.make_async_copy(
                bufs.at[b],
                out_hbm.at[pl.ds(wid * tok_per_w + j * TOK_CHUNK, TOK_CHUNK)],
                ssem.at[b])

        for b in range(NBUF):
            start_gathers(b, b)

        def body(g, carry):
            base = g * NBUF
            for b in range(NBUF):
                wait_gathers(b)
                scatter(base + b, b).start()
            for b in range(NBUF):
                scatter(base + b, b).wait()
                start_gathers(base + NBUF + b, b)
            return carry

        lax.fori_loop(0, num_groups - 1, body, 0)

        for b in range(NBUF):
            wait_gathers(b)
            scatter((num_groups - 1) * NBUF + b, b).start()
        for b in range(NBUF):
            scatter((num_groups - 1) * NBUF + b, b).wait()

    return gather_kernel


def kernel(token_ids, weight):
    num_tokens, seq = token_ids.shape
    dim = weight.shape[1]
    idx = token_ids.reshape(-1).astype(jnp.int32)
    info = plsc.get_sparse_core_info()
    nw = info.num_cores * info.num_subcores
    rows_per_w = idx.shape[0] // nw
    idx3 = idx.reshape(nw, rows_per_w // 128, 128)
    return _make_gather(num_tokens, seq, dim)(idx3, weight)


# speedup vs baseline: 1.4351x; 1.0141x over previous
import functools

import jax
import jax.numpy as jnp
from jax import lax
from jax.experimental import pallas as pl
from jax.experimental.pallas import tpu as pltpu
from jax.experimental.pallas import tpu_sc as plsc

TOK_CHUNK = 8
NBUF = 4


@functools.lru_cache(maxsize=None)
def _make_gather(num_tokens: int, seq: int, dim: int):
    info = plsc.get_sparse_core_info()
    nc, ns = info.num_cores, info.num_subcores
    nw = nc * ns
    tok_per_w = num_tokens // nw
    num_chunks = tok_per_w // TOK_CHUNK
    num_groups = num_chunks // NBUF
    assert num_chunks == num_groups * NBUF
    rows_per_chunk = TOK_CHUNK * seq
    rows_per_w = tok_per_w * seq
    assert rows_per_chunk % 16 == 0
    mesh = plsc.VectorSubcoreMesh(core_axis_name="c", subcore_axis_name="s")

    @functools.partial(
        pl.kernel,
        out_type=jax.ShapeDtypeStruct((num_tokens, seq, dim), jnp.float32),
        mesh=mesh,
        scratch_types=[
            pltpu.VMEM((rows_per_w // 128, 128), jnp.int32),
            pltpu.VMEM((NBUF, TOK_CHUNK, seq, dim), jnp.float32),
            pltpu.SemaphoreType.DMA((NBUF,)),
            pltpu.SemaphoreType.DMA((NBUF,)),
        ],
    )
    def gather_kernel(idx_hbm, table_hbm, out_hbm, idx_v, bufs, gsem, ssem):
        wid = lax.axis_index("c") * ns + lax.axis_index("s")
        pltpu.sync_copy(idx_hbm.at[wid], idx_v)

        def start_gathers(j, b):
            def inner(k, carry):
                flat = j * rows_per_chunk + k * 16
                vec = idx_v[flat // 128, pl.ds(lax.rem(flat, 128), 16)]
                for i in range(16):
                    r = flat + i
                    pltpu.make_async_copy(
                        table_hbm.at[vec[i]],
                        bufs.at[b, lax.rem(r, rows_per_chunk) // seq,
                                lax.rem(r, seq)],
                        gsem.at[b]).start()
                return carry
            lax.fori_loop(0, rows_per_chunk // 16, inner, 0)

        def wait_gathers(b):
            pltpu.make_async_copy(
                out_hbm.at[pl.ds(0, TOK_CHUNK)], bufs.at[b], gsem.at[b]).wait()

        def scatter(j, b):
            return pltpu.make_async_copy(
                bufs.at[b],
                out_hbm.at[pl.ds(wid * tok_per_w + j * TOK_CHUNK, TOK_CHUNK)],
                ssem.at[b])

        for b in range(NBUF):
            start_gathers(b, b)

        def body(g, carry):
            base = g * NBUF
            for b in range(NBUF):
                wait_gathers(b)
                scatter(base + b, b).start()
            for b in range(NBUF):
                scatter(base + b, b).wait()
                start_gathers(base + NBUF + b, b)
            return carry

        lax.fori_loop(0, num_groups - 1, body, 0)

        for b in range(NBUF):
            wait_gathers(b)
            scatter((num_groups - 1) * NBUF + b, b).start()
        for b in range(NBUF):
            scatter((num_groups - 1) * NBUF + b, b).wait()

    return gather_kernel


def kernel(token_ids, weight):
    num_tokens, seq = token_ids.shape
    dim = weight.shape[1]
    idx = token_ids.reshape(-1).astype(jnp.int32)
    info = plsc.get_sparse_core_info()
    nw = info.num_cores * info.num_subcores
    rows_per_w = idx.shape[0] // nw
    idx3 = idx.reshape(nw, rows_per_w // 128, 128)
    return _make_gather(num_tokens, seq, dim)(idx3, weight)


# speedup vs baseline: 1.4357x; 1.0005x over previous
import functools

import jax
import jax.numpy as jnp
from jax import lax
from jax.experimental import pallas as pl
from jax.experimental.pallas import tpu as pltpu
from jax.experimental.pallas import tpu_sc as plsc

TOK_CHUNK = 8
NBUF = 4


@functools.lru_cache(maxsize=None)
def _make_gather(num_tokens: int, seq: int, dim: int):
    info = plsc.get_sparse_core_info()
    nc, ns = info.num_cores, info.num_subcores
    nw = nc * ns
    tok_per_w = num_tokens // nw
    num_chunks = tok_per_w // TOK_CHUNK
    num_groups = num_chunks // NBUF
    assert num_chunks == num_groups * NBUF
    rows_per_chunk = TOK_CHUNK * seq
    rows_per_w = tok_per_w * seq
    assert rows_per_chunk % 16 == 0
    mesh = plsc.VectorSubcoreMesh(core_axis_name="c", subcore_axis_name="s")

    @functools.partial(
        pl.kernel,
        out_type=jax.ShapeDtypeStruct((num_tokens, seq, dim), jnp.float32),
        mesh=mesh,
        scratch_types=[
            pltpu.VMEM((rows_per_w // 128, 128), jnp.int32),
            pltpu.VMEM((NBUF, TOK_CHUNK, seq, dim), jnp.float32),
            pltpu.SemaphoreType.DMA((NBUF,)),
            pltpu.SemaphoreType.DMA((NBUF,)),
        ],
    )
    def gather_kernel(idx_hbm, table_hbm, out_hbm, idx_v, bufs, gsem, ssem):
        wid = lax.axis_index("c") * ns + lax.axis_index("s")
        pltpu.sync_copy(idx_hbm.at[wid], idx_v)

        def start_gathers(j, b):
            unit_rows = 4 * seq
            def inner(u, carry):
                flat = j * rows_per_chunk + u * unit_rows
                tbase = u * 4
                for v in range(unit_rows // 16):
                    off = flat + v * 16
                    vec = idx_v[off // 128, pl.ds(lax.rem(off, 128), 16)]
                    for i in range(16):
                        loc = v * 16 + i
                        pltpu.make_async_copy(
                            table_hbm.at[vec[i]],
                            bufs.at[b, tbase + loc // seq, loc % seq],
                            gsem.at[b]).start()
                return carry
            lax.fori_loop(0, rows_per_chunk // unit_rows, inner, 0)

        def wait_gathers(b):
            pltpu.make_async_copy(
                out_hbm.at[pl.ds(0, TOK_CHUNK)], bufs.at[b], gsem.at[b]).wait()

        def scatter(j, b):
            return pltpu.make_async_copy(
                bufs.at[b],
                out_hbm.at[pl.ds(wid * tok_per_w + j * TOK_CHUNK, TOK_CHUNK)],
                ssem.at[b])

        for b in range(NBUF):
            start_gathers(b, b)

        def body(g, carry):
            base = g * NBUF
            for b in range(NBUF):
                wait_gathers(b)
                scatter(base + b, b).start()
            for b in range(NBUF):
                scatter(base + b, b).wait()
                start_gathers(base + NBUF + b, b)
            return carry

        lax.fori_loop(0, num_groups - 1, body, 0)

        for b in range(NBUF):
            wait_gathers(b)
            scatter((num_groups - 1) * NBUF + b, b).start()
        for b in range(NBUF):
            scatter((num_groups - 1) * NBUF + b, b).wait()

    return gather_kernel


def kernel(token_ids, weight):
    num_tokens, seq = token_ids.shape
    dim = weight.shape[1]
    idx = token_ids.reshape(-1).astype(jnp.int32)
    info = plsc.get_sparse_core_info()
    nw = info.num_cores * info.num_subcores
    rows_per_w = idx.shape[0] // nw
    idx3 = idx.reshape(nw, rows_per_w // 128, 128)
    return _make_gather(num_tokens, seq, dim)(idx3, weight)


# speedup vs baseline: 1.4374x; 1.0012x over previous
import functools

import jax
import jax.numpy as jnp
from jax import lax
from jax.experimental import pallas as pl
from jax.experimental.pallas import tpu as pltpu
from jax.experimental.pallas import tpu_sc as plsc

TOK_CHUNK = 8
NBUF = 4


@functools.lru_cache(maxsize=None)
def _make_gather(num_tokens: int, seq: int, dim: int):
    info = plsc.get_sparse_core_info()
    nc, ns = info.num_cores, info.num_subcores
    nw = nc * ns
    tok_per_w = num_tokens // nw
    num_chunks = tok_per_w // TOK_CHUNK
    num_groups = num_chunks // NBUF
    assert num_chunks == num_groups * NBUF
    rows_per_chunk = TOK_CHUNK * seq
    rows_per_w = tok_per_w * seq
    assert rows_per_chunk % 16 == 0
    mesh = plsc.VectorSubcoreMesh(core_axis_name="c", subcore_axis_name="s")

    @functools.partial(
        pl.kernel,
        out_type=jax.ShapeDtypeStruct((num_tokens, seq, dim), jnp.float32),
        mesh=mesh,
        scratch_types=[
            pltpu.VMEM((rows_per_w // 128, 128), jnp.int32),
            pltpu.VMEM((NBUF, TOK_CHUNK, seq, dim), jnp.float32),
            pltpu.SemaphoreType.DMA((NBUF,)),
            pltpu.SemaphoreType.DMA((NBUF,)),
        ],
    )
    def gather_kernel(idx_hbm, table_hbm, out_hbm, idx_v, bufs, gsem, ssem):
        wid = lax.axis_index("c") * ns + lax.axis_index("s")
        pltpu.sync_copy(idx_hbm.at[wid], idx_v)

        def start_gathers(j, b):
            def inner(k, carry):
                flat = j * rows_per_chunk + k * 16
                vec = idx_v[flat // 128, pl.ds(lax.rem(flat, 128), 16)]
                for i in range(16):
                    r = flat + i
                    pltpu.make_async_copy(
                        table_hbm.at[vec[i]],
                        bufs.at[b, lax.rem(r, rows_per_chunk) // seq,
                                lax.rem(r, seq)],
                        gsem.at[b]).start()
                return carry
            lax.fori_loop(0, rows_per_chunk // 16, inner, 0)

        def wait_gathers(b):
            pltpu.make_async_copy(
                out_hbm.at[pl.ds(0, TOK_CHUNK)], bufs.at[b], gsem.at[b]).wait()

        def scatter(j, b):
            return pltpu.make_async_copy(
                bufs.at[b],
                out_hbm.at[pl.ds(wid * tok_per_w + j * TOK_CHUNK, TOK_CHUNK)],
                ssem.at[b])

        for b in range(NBUF):
            start_gathers(b, b)

        def body(g, carry):
            base = g * NBUF
            for b in range(NBUF):
                wait_gathers(b)
                scatter(base + b, b).start()
            for b in range(NBUF):
                scatter(base + b, b).wait()
                start_gathers(base + NBUF + b, b)
            return carry

        lax.fori_loop(0, num_groups - 1, body, 0)

        for b in range(NBUF):
            wait_gathers(b)
            scatter((num_groups - 1) * NBUF + b, b).start()
        for b in range(NBUF):
            scatter((num_groups - 1) * NBUF + b, b).wait()

    return gather_kernel


def kernel(token_ids, weight):
    num_tokens, seq = token_ids.shape
    dim = weight.shape[1]
    idx = token_ids.reshape(-1).astype(jnp.int32)
    info = plsc.get_sparse_core_info()
    nw = info.num_cores * info.num_subcores
    rows_per_w = idx.shape[0] // nw
    idx3 = idx.reshape(nw, rows_per_w // 128, 128)
    return _make_gather(num_tokens, seq, dim)(idx3, weight)
